# Initial kernel scaffold; baseline (speedup 1.0000x reference)
import jax
import jax.numpy as jnp
from jax.experimental import pallas as pl


def kernel(input_ids, target_ids, transition_logits):
    raise NotImplementedError("write your pallas kernel here")


# speedup vs baseline: 1.9968x; 1.9968x over previous
import functools

import jax
import jax.numpy as jnp
from jax import lax
from jax.experimental import pallas as pl
from jax.experimental.pallas import tpu as pltpu
from jax.experimental.pallas import tpu_sc as plsc

V = 8192
NTOK = 8192
ROWS_PER_BLK = 512
NBLK = V // ROWS_PER_BLK

NC = 2
NS = 16
L = 16
NW = NC * NS
TPW = NTOK // NW
CH = 128
NCH = TPW // CH
SUB = TPW // L
SUBPC = CH // L


def _lse_block(x_ref, o_ref):
    x = x_ref[...]
    m = jnp.max(x, axis=1)
    s = jnp.sum(jnp.exp(x - m[:, None]), axis=1)
    o_ref[...] = (m + jnp.log(s))[None, None, :]


def _compute_lse(table):
    out = pl.pallas_call(
        _lse_block,
        grid=(NBLK,),
        in_specs=[pl.BlockSpec((ROWS_PER_BLK, V), lambda i: (i, 0))],
        out_specs=pl.BlockSpec((1, 1, ROWS_PER_BLK), lambda i: (i, 0, 0)),
        out_shape=jax.ShapeDtypeStruct((NBLK, 1, ROWS_PER_BLK), jnp.float32),
    )(table)
    return out.reshape(V)


@functools.cache
def _sc_gather_kernel():
    mesh = plsc.VectorSubcoreMesh(
        core_axis_name="c", subcore_axis_name="s", num_cores=NC, num_subcores=NS
    )

    @functools.partial(
        pl.kernel,
        out_type=jax.ShapeDtypeStruct((NW, L), jnp.float32),
        mesh=mesh,
        scratch_types=[
            pltpu.VMEM((TPW,), jnp.int32),
            pltpu.VMEM((TPW,), jnp.int32),
            pltpu.VMEM((NCH, CH), jnp.int32),
            pltpu.VMEM((NCH, CH, L), jnp.float32),
            pltpu.VMEM((V,), jnp.float32),
            pltpu.VMEM((L,), jnp.float32),
            pltpu.SemaphoreType.DMA,
        ],
        compiler_params=pltpu.CompilerParams(
            use_tc_tiling_on_sc=False, needs_layout_passes=False
        ),
    )
    def _sc_gather(prev_hbm, tgt_hbm, lse_hbm, tblk_hbm, out_hbm,
                   prev_v, tgt_v, blk_v, row_v, lse_v, acc_v, sem):
        _sc_gather_body(prev_hbm, tgt_hbm, lse_hbm, tblk_hbm, out_hbm,
                        prev_v, tgt_v, blk_v, row_v, lse_v, acc_v, sem)

    return _sc_gather


def _sc_gather_body(prev_hbm, tgt_hbm, lse_hbm, tblk_hbm, out_hbm,
                    prev_v, tgt_v, blk_v, row_v, lse_v, acc_v, sem):
    wid = lax.axis_index("s") * NC + lax.axis_index("c")
    base = wid * TPW
    pltpu.sync_copy(prev_hbm.at[pl.ds(base, TPW)], prev_v)
    pltpu.sync_copy(tgt_hbm.at[pl.ds(base, TPW)], tgt_v)
    pltpu.sync_copy(lse_hbm, lse_v)

    for j in range(SUB):
        p = prev_v[pl.ds(j * L, L)]
        t = tgt_v[pl.ds(j * L, L)]
        blk = p * (V // L) + lax.shift_right_logical(t, 4)
        blk_v[j // SUBPC, pl.ds((j % SUBPC) * L, L)] = blk

    copies = [
        pltpu.async_copy(tblk_hbm.at[blk_v.at[c]], row_v.at[c], sem)
        for c in range(NCH)
    ]
    for cp in copies:
        cp.wait()

    acc = jnp.zeros((L,), jnp.float32)
    iota = lax.iota(jnp.int32, L)
    for j in range(SUB):
        p = prev_v[pl.ds(j * L, L)]
        t = tgt_v[pl.ds(j * L, L)]
        lse_g = plsc.load_gather(lse_v, [p])
        rows = iota + (j % SUBPC) * L
        offs = jnp.bitwise_and(t, L - 1)
        vals = plsc.load_gather(row_v.at[j // SUBPC], [rows, offs])
        acc = acc + (lse_g - vals)
    acc_v[...] = acc
    pltpu.sync_copy(acc_v, out_hbm.at[wid])


def kernel(input_ids, target_ids, transition_logits):
    prev = input_ids.reshape(-1).astype(jnp.int32)
    tgt = target_ids.reshape(-1).astype(jnp.int32)
    table = transition_logits.astype(jnp.float32)
    lse = _compute_lse(table)
    tblk = table.reshape(V * V // L, L)
    partials = _sc_gather_kernel()(prev, tgt, lse, tblk)
    return jnp.sum(partials) / NTOK
